# Initial kernel scaffold; baseline (speedup 1.0000x reference)
import jax
import jax.numpy as jnp
from jax.experimental import pallas as pl


def kernel(residue_ca_pos, residue_mask, atom_pos, atom_mask, max_p):
    raise NotImplementedError("write your pallas kernel here")


# speedup vs baseline: 8.8108x; 8.8108x over previous
import numpy as np
import jax
import jax.numpy as jnp
from jax.experimental import pallas as pl

_INF = 10000000000.0


def _body(k, ca_ref, at_ref, rm_ref, am_ref, sp_ref, esm_ref):
    ca = ca_ref[0]
    at = at_ref[0]
    rm = rm_ref[0]
    am = am_ref[0]

    inv = 1.0 / jnp.sum(am)
    cx = jnp.sum(at[0]) * inv
    cy = jnp.sum(at[1]) * inv
    cz = jnp.sum(at[2]) * inv

    dx = ca[0] - cx
    dy = ca[1] - cy
    dz = ca[2] - cz
    key = dx * dx + dy * dy + dz * dz + (1.0 - rm) * _INF

    bits = jax.lax.bitcast_convert_type(key, jnp.int32)

    def step(i, t):
        cand = jnp.bitwise_or(t, jax.lax.shift_left(jnp.int32(1), 30 - i))
        cnt = jnp.sum((bits < cand).astype(jnp.int32))
        return jnp.where(cnt >= k, t, cand)

    thr = jax.lax.fori_loop(0, 31, step, jnp.int32(0))
    sel = bits <= thr

    sp_ref[0] = jnp.where(sel, 0.0, rm)
    esm_ref[0] = jnp.where(sel, 32.0, 1.0 - rm)


def kernel(residue_ca_pos, residue_mask, atom_pos, atom_mask, max_p):
    B, N, _ = residue_ca_pos.shape
    n_mean_res = float(residue_mask.shape[-1])
    np.random.seed(0)
    top_k = int(np.random.choice(np.linspace(0, 1, 1000)) * n_mean_res)
    top_k = max(top_k, 1)

    L = 128
    R = N // L
    ca = jnp.transpose(residue_ca_pos, (0, 2, 1)).reshape(B, 3, R, L)
    at = jnp.transpose(atom_pos, (0, 2, 1)).reshape(B, 3, R, L)
    rm = residue_mask.reshape(B, R, L)
    am = atom_mask.reshape(B, R, L)

    spatial, esm = pl.pallas_call(
        lambda *refs: _body(top_k, *refs),
        grid=(B,),
        in_specs=[
            pl.BlockSpec((1, 3, R, L), lambda b: (b, 0, 0, 0)),
            pl.BlockSpec((1, 3, R, L), lambda b: (b, 0, 0, 0)),
            pl.BlockSpec((1, R, L), lambda b: (b, 0, 0)),
            pl.BlockSpec((1, R, L), lambda b: (b, 0, 0)),
        ],
        out_specs=[
            pl.BlockSpec((1, R, L), lambda b: (b, 0, 0)),
            pl.BlockSpec((1, R, L), lambda b: (b, 0, 0)),
        ],
        out_shape=[
            jax.ShapeDtypeStruct((B, R, L), jnp.float32),
            jax.ShapeDtypeStruct((B, R, L), jnp.float32),
        ],
    )(ca, at, rm, am)

    return (spatial.reshape(B, N), esm.reshape(B, N))
